# Initial kernel scaffold; baseline (speedup 1.0000x reference)
import jax
import jax.numpy as jnp
from jax import lax
from jax.experimental import pallas as pl
from jax.experimental.pallas import tpu as pltpu
from jax.experimental.pallas import tpu_sc as plsc

N_PED = 320000
N_GROUPS = 10000
N_FEAT = 60
NUM_CORES = 2
NUM_SUBCORES = 16
LANES = 16


def _build(n_ped, n_groups, n_feat, interpret=False):
  nw = NUM_CORES * NUM_SUBCORES
  chunk = n_ped // nw
  nv = chunk // LANES
  grow_used = (n_groups + LANES - 1) // LANES
  grow = ((grow_used + 15) // 16) * 16 + 16
  nrow_loc = grow + 16
  acc_rows = (n_feat + 1) * grow
  f_per_s = (n_feat + 1 + NUM_SUBCORES - 1) // NUM_SUBCORES

  mesh = plsc.VectorSubcoreMesh(core_axis_name="c", subcore_axis_name="s")
  zvec = jnp.zeros((LANES,), jnp.float32)

  def pool_body(v_hbm, idx_hbm, out_hbm, idxb, datab, lacc, acc_sh):
    c = lax.axis_index("c")
    s = lax.axis_index("s")
    wid = s * NUM_CORES + c
    base = wid * chunk
    iota = lax.iota(jnp.int32, LANES)

    def zl(r, carry):
      lacc[r] = zvec
      return carry

    lax.fori_loop(0, nrow_loc, zl, 0)
    for k in range(f_per_s):
      fs = s * f_per_s + k

      @pl.when(fs < n_feat + 1)
      def _():
        pltpu.sync_copy(
            lacc.at[pl.ds(0, grow)], acc_sh.at[pl.ds(fs * grow, grow)]
        )

    plsc.subcore_barrier()

    pltpu.sync_copy(idx_hbm.at[pl.ds(base, chunk)], idxb)
    lo = idxb[0]
    hi = idxb[chunk - 1]
    row_lo = lax.shift_right_logical(lo, 4)
    row_hi = lax.shift_right_logical(hi, 4)
    ndma = lax.shift_right_logical(row_hi - row_lo, 4) + 1

    def accum_feature(f, have_data):
      def zr(j, carry):
        lacc[row_lo + j] = zvec
        return carry

      lax.fori_loop(0, ndma * 16, zr, 0)
      ones = jnp.ones((LANES,), jnp.float32)

      def body(i, carry):
        idxv = idxb[pl.ds(i * LANES, LANES)]
        d = datab[pl.ds(i * LANES, LANES)] if have_data else ones
        plsc.addupdate_scatter(
            lacc,
            [lax.shift_right_logical(idxv, 4), lax.bitwise_and(idxv, 15)],
            d,
        )
        return carry

      lax.fori_loop(0, nv, body, 0)

      def dma(j, carry):
        r0 = row_lo + j * 16
        rows = f * grow + r0 + iota
        pltpu.sync_copy(lacc.at[pl.ds(r0, 16)], acc_sh.at[rows], add=True)
        return carry

      lax.fori_loop(0, ndma, dma, 0)

    accum_feature(n_feat, False)

    def fbody(f, carry):
      pltpu.sync_copy(v_hbm.at[f, pl.ds(base, chunk)], datab)
      accum_feature(f, True)
      return carry

    lax.fori_loop(0, n_feat, fbody, 0)
    plsc.subcore_barrier()

    for k in range(f_per_s):
      fs = s * f_per_s + k

      @pl.when(fs < n_feat + 1)
      def _():
        pltpu.sync_copy(
            acc_sh.at[pl.ds(fs * grow, grow)],
            out_hbm.at[c, pl.ds(fs * grow, grow)],
        )

  pool = pl.kernel(
      pool_body,
      out_type=jax.ShapeDtypeStruct((NUM_CORES, acc_rows, LANES), jnp.float32),
      mesh=mesh,
      scratch_types=[
          pltpu.VMEM((chunk,), jnp.int32),
          pltpu.VMEM((chunk,), jnp.float32),
          pltpu.VMEM((nrow_loc, LANES), jnp.float32),
          pltpu.VMEM_SHARED((acc_rows, LANES), jnp.float32),
      ],
      interpret=interpret,
  )

  def unpool_body(part_hbm, idx_hbm, out_hbm, idxb, a0, a1, icnt, pooled, outb):
    c = lax.axis_index("c")
    s = lax.axis_index("s")
    wid = s * NUM_CORES + c
    base = wid * chunk

    pltpu.sync_copy(idx_hbm.at[pl.ds(base, chunk)], idxb)
    lo = idxb[0]
    hi = idxb[chunk - 1]
    row_lo = lax.shift_right_logical(lo, 4)
    row_hi = lax.shift_right_logical(hi, 4)
    ndma = lax.shift_right_logical(row_hi - row_lo, 4) + 1

    def ld_rows(f):
      def dj(j, carry):
        r0 = row_lo + j * 16
        pltpu.sync_copy(
            part_hbm.at[0, pl.ds(f * grow + r0, 16)], a0.at[pl.ds(r0, 16)]
        )
        pltpu.sync_copy(
            part_hbm.at[1, pl.ds(f * grow + r0, 16)], a1.at[pl.ds(r0, 16)]
        )
        return carry

      lax.fori_loop(0, ndma, dj, 0)

    ld_rows(n_feat)

    def ci(r, carry):
      cv = a0[r] + a1[r]
      icnt[r] = 1.0 / jnp.maximum(cv, 1.0)
      return carry

    lax.fori_loop(row_lo, row_lo + ndma * 16, ci, 0)

    def fbody(f, carry):
      ld_rows(f)

      def pr(r, c2):
        pooled[r] = (a0[r] + a1[r]) * icnt[r]
        return c2

      lax.fori_loop(row_lo, row_lo + ndma * 16, pr, 0)

      def gb(i, c2):
        idxv = idxb[pl.ds(i * LANES, LANES)]
        o = plsc.load_gather(
            pooled,
            [lax.shift_right_logical(idxv, 4), lax.bitwise_and(idxv, 15)],
        )
        outb[pl.ds(i * LANES, LANES)] = o
        return c2

      lax.fori_loop(0, nv, gb, 0)
      pltpu.sync_copy(outb, out_hbm.at[f, pl.ds(base, chunk)])
      return carry

    lax.fori_loop(0, n_feat, fbody, 0)

  unpool = pl.kernel(
      unpool_body,
      out_type=jax.ShapeDtypeStruct((n_feat, n_ped), jnp.float32),
      mesh=mesh,
      scratch_types=[
          pltpu.VMEM((chunk,), jnp.int32),
          pltpu.VMEM((nrow_loc, LANES), jnp.float32),
          pltpu.VMEM((nrow_loc, LANES), jnp.float32),
          pltpu.VMEM((nrow_loc, LANES), jnp.float32),
          pltpu.VMEM((nrow_loc, LANES), jnp.float32),
          pltpu.VMEM((chunk,), jnp.float32),
      ],
      interpret=interpret,
  )

  def run(v, indices):
    n_feat_v = v.shape[1] * v.shape[2]
    v2d = v.reshape(n_feat_v, v.shape[3])
    idx = indices.astype(jnp.int32)
    part = pool(v2d, idx)
    out2d = unpool(part, idx)
    return out2d.reshape(v.shape)

  return run


_run = _build(N_PED, N_GROUPS, N_FEAT)


@jax.jit
def kernel(v, indices):
  return _run(v, indices)


# speedup vs baseline: 2.0223x; 2.0223x over previous
import jax
import jax.numpy as jnp
from jax import lax
from jax.experimental import pallas as pl
from jax.experimental.pallas import tpu as pltpu
from jax.experimental.pallas import tpu_sc as plsc

N_PED = 320000
N_GROUPS = 10000
N_FEAT = 60
NUM_CORES = 2
NUM_SUBCORES = 16
LANES = 16


def _build(n_ped, n_groups, n_feat, interpret=False):
  nw = NUM_CORES * NUM_SUBCORES
  chunk = n_ped // nw
  nv = chunk // LANES
  grow_used = (n_groups + LANES - 1) // LANES
  grow = ((grow_used + 15) // 16) * 16 + 16
  nrow_loc = grow + 16
  acc_rows = (n_feat + 1) * grow
  f_per_s = (n_feat + 1 + NUM_SUBCORES - 1) // NUM_SUBCORES

  mesh = plsc.VectorSubcoreMesh(
      core_axis_name="c",
      subcore_axis_name="s",
      num_cores=NUM_CORES,
      num_subcores=NUM_SUBCORES,
  )

  def pool_body(v_hbm, idx_hbm, out_hbm, idxb, datab, lacc, acc_sh):
    c = lax.axis_index("c")
    s = lax.axis_index("s")
    wid = s * NUM_CORES + c
    base = pl.multiple_of(wid * chunk, 512)
    iota = lax.iota(jnp.int32, LANES)
    zvec = jnp.zeros((LANES,), jnp.float32)

    def zl(r, carry):
      lacc[r] = zvec
      return carry

    lax.fori_loop(0, nrow_loc, zl, 0)
    for k in range(f_per_s):
      fs = s * f_per_s + k

      @pl.when(fs < n_feat + 1)
      def _():
        pltpu.sync_copy(
            lacc.at[pl.ds(0, grow)], acc_sh.at[pl.ds(fs * grow, grow)]
        )

    plsc.subcore_barrier()

    pltpu.sync_copy(idx_hbm.at[pl.ds(base, chunk)], idxb)
    lo = idxb[pl.ds(0, LANES)][0]
    hi = idxb[pl.ds(chunk - LANES, LANES)][LANES - 1]
    row_lo = lax.shift_right_logical(lo, 4)
    row_hi = lax.shift_right_logical(hi, 4)
    ndma = lax.shift_right_logical(row_hi - row_lo, 4) + 1

    def accum_feature(f, have_data):
      def zr(j, carry):
        lacc[row_lo + j] = zvec
        return carry

      lax.fori_loop(0, ndma * 16, zr, 0)
      ones = jnp.ones((LANES,), jnp.float32)

      def body(i, carry):
        idxv = idxb[pl.ds(i * LANES, LANES)]
        d = datab[pl.ds(i * LANES, LANES)] if have_data else ones
        plsc.addupdate_scatter(
            lacc,
            [lax.shift_right_logical(idxv, 4), lax.bitwise_and(idxv, 15)],
            d,
        )
        return carry

      lax.fori_loop(0, nv, body, 0)

      def dma(j, carry):
        r0 = row_lo + j * 16
        rows = f * grow + r0 + iota
        pltpu.sync_copy(lacc.at[pl.ds(r0, 16)], acc_sh.at[rows], add=True)
        return carry

      lax.fori_loop(0, ndma, dma, 0)

    accum_feature(n_feat, False)

    def fbody(f, carry):
      pltpu.sync_copy(v_hbm.at[f, pl.ds(base, chunk)], datab)
      accum_feature(f, True)
      return carry

    lax.fori_loop(0, n_feat, fbody, 0)
    plsc.subcore_barrier()

    for k in range(f_per_s):
      fs = s * f_per_s + k

      @pl.when(fs < n_feat + 1)
      def _():
        pltpu.sync_copy(
            acc_sh.at[pl.ds(fs * grow, grow)],
            out_hbm.at[c, pl.ds(fs * grow, grow)],
        )

  pool = pl.kernel(
      pool_body,
      out_type=jax.ShapeDtypeStruct((NUM_CORES, acc_rows, LANES), jnp.float32),
      mesh=mesh,
      scratch_types=[
          pltpu.VMEM((chunk,), jnp.int32),
          pltpu.VMEM((chunk,), jnp.float32),
          pltpu.VMEM((nrow_loc, LANES), jnp.float32),
          pltpu.VMEM_SHARED((acc_rows, LANES), jnp.float32),
      ],
      compiler_params=pltpu.CompilerParams(use_tc_tiling_on_sc=False, needs_layout_passes=False),
      interpret=interpret,
  )

  def unpool_body(part_hbm, idx_hbm, out_hbm, idxb, a0, a1, icnt, pooled, outb):
    c = lax.axis_index("c")
    s = lax.axis_index("s")
    wid = s * NUM_CORES + c
    base = pl.multiple_of(wid * chunk, 512)

    pltpu.sync_copy(idx_hbm.at[pl.ds(base, chunk)], idxb)
    lo = idxb[pl.ds(0, LANES)][0]
    hi = idxb[pl.ds(chunk - LANES, LANES)][LANES - 1]
    row_lo = lax.shift_right_logical(lo, 4)
    row_hi = lax.shift_right_logical(hi, 4)
    ndma = lax.shift_right_logical(row_hi - row_lo, 4) + 1

    def ld_rows(f):
      def dj(j, carry):
        r0 = row_lo + j * 16
        pltpu.sync_copy(
            part_hbm.at[0, pl.ds(f * grow + r0, 16)], a0.at[pl.ds(r0, 16)]
        )
        pltpu.sync_copy(
            part_hbm.at[1, pl.ds(f * grow + r0, 16)], a1.at[pl.ds(r0, 16)]
        )
        return carry

      lax.fori_loop(0, ndma, dj, 0)

    ld_rows(n_feat)

    def ci(r, carry):
      cv = a0[r] + a1[r]
      icnt[r] = 1.0 / jnp.maximum(cv, 1.0)
      return carry

    lax.fori_loop(row_lo, row_lo + ndma * 16, ci, 0)

    def fbody(f, carry):
      ld_rows(f)

      def pr(r, c2):
        pooled[r] = (a0[r] + a1[r]) * icnt[r]
        return c2

      lax.fori_loop(row_lo, row_lo + ndma * 16, pr, 0)

      def gb(i, c2):
        idxv = idxb[pl.ds(i * LANES, LANES)]
        o = plsc.load_gather(
            pooled,
            [lax.shift_right_logical(idxv, 4), lax.bitwise_and(idxv, 15)],
        )
        outb[pl.ds(i * LANES, LANES)] = o
        return c2

      lax.fori_loop(0, nv, gb, 0)
      pltpu.sync_copy(outb, out_hbm.at[f, pl.ds(base, chunk)])
      return carry

    lax.fori_loop(0, n_feat, fbody, 0)

  unpool = pl.kernel(
      unpool_body,
      out_type=jax.ShapeDtypeStruct((n_feat, n_ped), jnp.float32),
      mesh=mesh,
      scratch_types=[
          pltpu.VMEM((chunk,), jnp.int32),
          pltpu.VMEM((nrow_loc, LANES), jnp.float32),
          pltpu.VMEM((nrow_loc, LANES), jnp.float32),
          pltpu.VMEM((nrow_loc, LANES), jnp.float32),
          pltpu.VMEM((nrow_loc, LANES), jnp.float32),
          pltpu.VMEM((chunk,), jnp.float32),
      ],
      compiler_params=pltpu.CompilerParams(use_tc_tiling_on_sc=False, needs_layout_passes=False),
      interpret=interpret,
  )

  def run(v, indices):
    n_feat_v = v.shape[1] * v.shape[2]
    v2d = v.reshape(n_feat_v, v.shape[3])
    idx = indices.astype(jnp.int32)
    part = pool(v2d, idx)
    out2d = unpool(part, idx)
    return out2d.reshape(v.shape)

  return run


_run = _build(N_PED, N_GROUPS, N_FEAT)


@jax.jit
def kernel(v, indices):
  return _run(v, indices)


# speedup vs baseline: 2.1672x; 1.0717x over previous
import jax
import jax.numpy as jnp
from jax import lax
from jax.experimental import pallas as pl
from jax.experimental.pallas import tpu as pltpu
from jax.experimental.pallas import tpu_sc as plsc

N_PED = 320000
N_GROUPS = 10000
N_FEAT = 60
NUM_CORES = 2
NUM_SUBCORES = 16
LANES = 16
UNROLL = 5


def _build(n_ped, n_groups, n_feat, interpret=False):
  nw = NUM_CORES * NUM_SUBCORES
  chunk = n_ped // nw
  nv = chunk // LANES
  assert nv % UNROLL == 0
  grow_used = (n_groups + LANES - 1) // LANES
  grow = ((grow_used + 15) // 16) * 16 + 16
  nrow_loc = grow + 16
  acc_rows = (n_feat + 1) * grow
  f_per_s = (n_feat + 1 + NUM_SUBCORES - 1) // NUM_SUBCORES

  mesh = plsc.VectorSubcoreMesh(
      core_axis_name="c",
      subcore_axis_name="s",
      num_cores=NUM_CORES,
      num_subcores=NUM_SUBCORES,
  )

  def pool_body(v_hbm, idx_hbm, out_hbm, idxb, datab, lacc, acc_sh):
    c = lax.axis_index("c")
    s = lax.axis_index("s")
    wid = s * NUM_CORES + c
    base = pl.multiple_of(wid * chunk, 512)
    iota = lax.iota(jnp.int32, LANES)
    zvec = jnp.zeros((LANES,), jnp.float32)

    def zl(r, carry):
      lacc[r] = zvec
      return carry

    lax.fori_loop(0, nrow_loc, zl, 0)
    for k in range(f_per_s):
      fs = s * f_per_s + k

      @pl.when(fs < n_feat + 1)
      def _():
        pltpu.sync_copy(
            lacc.at[pl.ds(0, grow)], acc_sh.at[pl.ds(fs * grow, grow)]
        )

    plsc.subcore_barrier()

    pltpu.sync_copy(idx_hbm.at[pl.ds(base, chunk)], idxb)
    lo = idxb[pl.ds(0, LANES)][0]
    hi = idxb[pl.ds(chunk - LANES, LANES)][LANES - 1]
    row_lo = lax.shift_right_logical(lo, 4)
    row_hi = lax.shift_right_logical(hi, 4)
    ndma = lax.shift_right_logical(row_hi - row_lo, 4) + 1

    def accum_feature(f, have_data):
      def zr(j, carry):
        r0 = row_lo + j * 16
        for k in range(16):
          lacc[r0 + k] = zvec
        return carry

      lax.fori_loop(0, ndma, zr, 0)
      ones = jnp.ones((LANES,), jnp.float32)

      def body(i, carry):
        for k in range(UNROLL):
          off = (i * UNROLL + k) * LANES
          idxv = idxb[pl.ds(off, LANES)]
          d = datab[pl.ds(off, LANES)] if have_data else ones
          plsc.addupdate_scatter(
              lacc,
              [lax.shift_right_logical(idxv, 4), lax.bitwise_and(idxv, 15)],
              d,
          )
        return carry

      lax.fori_loop(0, nv // UNROLL, body, 0)

      def dma(j, carry):
        r0 = row_lo + j * 16
        rows = f * grow + r0 + iota
        pltpu.sync_copy(lacc.at[pl.ds(r0, 16)], acc_sh.at[rows], add=True)
        return carry

      lax.fori_loop(0, ndma, dma, 0)

    accum_feature(n_feat, False)

    def fbody(f, carry):
      pltpu.sync_copy(v_hbm.at[f, pl.ds(base, chunk)], datab)
      accum_feature(f, True)
      return carry

    lax.fori_loop(0, n_feat, fbody, 0)
    plsc.subcore_barrier()

    for k in range(f_per_s):
      fs = s * f_per_s + k

      @pl.when(fs < n_feat + 1)
      def _():
        pltpu.sync_copy(
            acc_sh.at[pl.ds(fs * grow, grow)],
            out_hbm.at[c, pl.ds(fs * grow, grow)],
        )

  pool = pl.kernel(
      pool_body,
      out_type=jax.ShapeDtypeStruct((NUM_CORES, acc_rows, LANES), jnp.float32),
      mesh=mesh,
      scratch_types=[
          pltpu.VMEM((chunk,), jnp.int32),
          pltpu.VMEM((chunk,), jnp.float32),
          pltpu.VMEM((nrow_loc, LANES), jnp.float32),
          pltpu.VMEM_SHARED((acc_rows, LANES), jnp.float32),
      ],
      compiler_params=pltpu.CompilerParams(use_tc_tiling_on_sc=False, needs_layout_passes=False),
      interpret=interpret,
  )

  def unpool_body(part_hbm, idx_hbm, out_hbm, idxb, a0, a1, icnt, pooled, outb):
    c = lax.axis_index("c")
    s = lax.axis_index("s")
    wid = s * NUM_CORES + c
    base = pl.multiple_of(wid * chunk, 512)

    pltpu.sync_copy(idx_hbm.at[pl.ds(base, chunk)], idxb)
    lo = idxb[pl.ds(0, LANES)][0]
    hi = idxb[pl.ds(chunk - LANES, LANES)][LANES - 1]
    row_lo = lax.shift_right_logical(lo, 4)
    row_hi = lax.shift_right_logical(hi, 4)
    ndma = lax.shift_right_logical(row_hi - row_lo, 4) + 1

    def ld_rows(f):
      def dj(j, carry):
        r0 = row_lo + j * 16
        pltpu.sync_copy(
            part_hbm.at[0, pl.ds(f * grow + r0, 16)], a0.at[pl.ds(r0, 16)]
        )
        pltpu.sync_copy(
            part_hbm.at[1, pl.ds(f * grow + r0, 16)], a1.at[pl.ds(r0, 16)]
        )
        return carry

      lax.fori_loop(0, ndma, dj, 0)

    ld_rows(n_feat)

    def ci(j, carry):
      r0 = row_lo + j * 16
      for k in range(16):
        cv = a0[r0 + k] + a1[r0 + k]
        icnt[r0 + k] = 1.0 / jnp.maximum(cv, 1.0)
      return carry

    lax.fori_loop(0, ndma, ci, 0)

    def fbody(f, carry):
      ld_rows(f)

      def pr(j, c2):
        r0 = row_lo + j * 16
        for k in range(16):
          pooled[r0 + k] = (a0[r0 + k] + a1[r0 + k]) * icnt[r0 + k]
        return c2

      lax.fori_loop(0, ndma, pr, 0)

      def gb(i, c2):
        for k in range(UNROLL):
          off = (i * UNROLL + k) * LANES
          idxv = idxb[pl.ds(off, LANES)]
          o = plsc.load_gather(
              pooled,
              [lax.shift_right_logical(idxv, 4), lax.bitwise_and(idxv, 15)],
          )
          outb[pl.ds(off, LANES)] = o
        return c2

      lax.fori_loop(0, nv // UNROLL, gb, 0)
      pltpu.sync_copy(outb, out_hbm.at[f, pl.ds(base, chunk)])
      return carry

    lax.fori_loop(0, n_feat, fbody, 0)

  unpool = pl.kernel(
      unpool_body,
      out_type=jax.ShapeDtypeStruct((n_feat, n_ped), jnp.float32),
      mesh=mesh,
      scratch_types=[
          pltpu.VMEM((chunk,), jnp.int32),
          pltpu.VMEM((nrow_loc, LANES), jnp.float32),
          pltpu.VMEM((nrow_loc, LANES), jnp.float32),
          pltpu.VMEM((nrow_loc, LANES), jnp.float32),
          pltpu.VMEM((nrow_loc, LANES), jnp.float32),
          pltpu.VMEM((chunk,), jnp.float32),
      ],
      compiler_params=pltpu.CompilerParams(use_tc_tiling_on_sc=False, needs_layout_passes=False),
      interpret=interpret,
  )

  def run(v, indices):
    n_feat_v = v.shape[1] * v.shape[2]
    v2d = v.reshape(n_feat_v, v.shape[3])
    idx = indices.astype(jnp.int32)
    part = pool(v2d, idx)
    out2d = unpool(part, idx)
    return out2d.reshape(v.shape)

  return run


_run = _build(N_PED, N_GROUPS, N_FEAT)


@jax.jit
def kernel(v, indices):
  return _run(v, indices)


# speedup vs baseline: 2.5791x; 1.1900x over previous
import jax
import jax.numpy as jnp
from jax import lax
from jax.experimental import pallas as pl
from jax.experimental.pallas import tpu as pltpu
from jax.experimental.pallas import tpu_sc as plsc

N_PED = 320000
N_GROUPS = 10000
N_FEAT = 60
NUM_CORES = 2
NUM_SUBCORES = 16
LANES = 16
UNROLL = 5
NPLANES = 4


def _build(n_ped, n_groups, n_feat, interpret=False):
  nw = NUM_CORES * NUM_SUBCORES
  chunk = n_ped // nw
  nv = chunk // LANES
  assert nv % UNROLL == 0
  grow_used = (n_groups + LANES - 1) // LANES
  grow = ((grow_used + 15) // 16) * 16 + 16
  nrow_loc = grow + 16
  acc_rows = (n_feat + 1) * grow
  f_per_s = (n_feat + 1 + NUM_SUBCORES - 1) // NUM_SUBCORES

  mesh = plsc.VectorSubcoreMesh(
      core_axis_name="c",
      subcore_axis_name="s",
      num_cores=NUM_CORES,
      num_subcores=NUM_SUBCORES,
  )

  gpad = nrow_loc * LANES

  def pool_body(v_hbm, idx_hbm, out_hbm, idxb, datab, planes, rowbuf, acc_sh):
    c = lax.axis_index("c")
    s = lax.axis_index("s")
    wid = s * NUM_CORES + c
    base = pl.multiple_of(wid * chunk, 512)
    iota = lax.iota(jnp.int32, LANES)
    zvec = jnp.zeros((LANES,), jnp.float32)
    pbase = lax.bitwise_and(iota, NPLANES - 1) * gpad

    def zp(j, carry):
      for k in range(8):
        planes[pl.ds((j * 8 + k) * LANES, LANES)] = zvec
      return carry

    lax.fori_loop(0, NPLANES * gpad // (8 * LANES), zp, 0)

    def zl(r, carry):
      rowbuf[r] = zvec
      return carry

    lax.fori_loop(0, nrow_loc, zl, 0)
    for k in range(f_per_s):
      fs = s * f_per_s + k

      @pl.when(fs < n_feat + 1)
      def _():
        pltpu.sync_copy(
            rowbuf.at[pl.ds(0, grow)], acc_sh.at[pl.ds(fs * grow, grow)]
        )

    plsc.subcore_barrier()

    pltpu.sync_copy(idx_hbm.at[pl.ds(base, chunk)], idxb)
    lo = idxb[pl.ds(0, LANES)][0]
    hi = idxb[pl.ds(chunk - LANES, LANES)][LANES - 1]
    row_lo = lax.shift_right_logical(lo, 4)
    row_hi = lax.shift_right_logical(hi, 4)
    ndma = lax.shift_right_logical(row_hi - row_lo, 4) + 1

    def accum_feature(f, have_data):
      ones = jnp.ones((LANES,), jnp.float32)

      def body(i, carry):
        for k in range(UNROLL):
          off = (i * UNROLL + k) * LANES
          idxv = idxb[pl.ds(off, LANES)]
          d = datab[pl.ds(off, LANES)] if have_data else ones
          plsc.addupdate_scatter(planes, [pbase + idxv], d)
        return carry

      lax.fori_loop(0, nv // UNROLL, body, 0)

      def red(j, carry):
        r0 = row_lo + j * 16
        for k in range(16):
          r = r0 + k
          off = r * LANES
          acc = planes[pl.ds(off, LANES)]
          for p in range(1, NPLANES):
            acc = acc + planes[pl.ds(p * gpad + off, LANES)]
          for p in range(NPLANES):
            planes[pl.ds(p * gpad + off, LANES)] = zvec
          rowbuf[r] = acc
        return carry

      lax.fori_loop(0, ndma, red, 0)

      def dma(j, carry):
        r0 = row_lo + j * 16
        rows = f * grow + r0 + iota
        pltpu.sync_copy(rowbuf.at[pl.ds(r0, 16)], acc_sh.at[rows], add=True)
        return carry

      lax.fori_loop(0, ndma, dma, 0)

    accum_feature(n_feat, False)

    def fbody(f, carry):
      pltpu.sync_copy(v_hbm.at[f, pl.ds(base, chunk)], datab)
      accum_feature(f, True)
      return carry

    lax.fori_loop(0, n_feat, fbody, 0)
    plsc.subcore_barrier()

    for k in range(f_per_s):
      fs = s * f_per_s + k

      @pl.when(fs < n_feat + 1)
      def _():
        pltpu.sync_copy(
            acc_sh.at[pl.ds(fs * grow, grow)],
            out_hbm.at[c, pl.ds(fs * grow, grow)],
        )

  pool = pl.kernel(
      pool_body,
      out_type=jax.ShapeDtypeStruct((NUM_CORES, acc_rows, LANES), jnp.float32),
      mesh=mesh,
      scratch_types=[
          pltpu.VMEM((chunk,), jnp.int32),
          pltpu.VMEM((chunk,), jnp.float32),
          pltpu.VMEM((NPLANES * nrow_loc * LANES,), jnp.float32),
          pltpu.VMEM((nrow_loc, LANES), jnp.float32),
          pltpu.VMEM_SHARED((acc_rows, LANES), jnp.float32),
      ],
      compiler_params=pltpu.CompilerParams(use_tc_tiling_on_sc=False, needs_layout_passes=False),
      interpret=interpret,
  )

  def unpool_body(part_hbm, idx_hbm, out_hbm, idxb, a0, a1, icnt, pooled, outb):
    c = lax.axis_index("c")
    s = lax.axis_index("s")
    wid = s * NUM_CORES + c
    base = pl.multiple_of(wid * chunk, 512)

    pltpu.sync_copy(idx_hbm.at[pl.ds(base, chunk)], idxb)
    lo = idxb[pl.ds(0, LANES)][0]
    hi = idxb[pl.ds(chunk - LANES, LANES)][LANES - 1]
    row_lo = lax.shift_right_logical(lo, 4)
    row_hi = lax.shift_right_logical(hi, 4)
    ndma = lax.shift_right_logical(row_hi - row_lo, 4) + 1

    def ld_rows(f):
      def dj(j, carry):
        r0 = row_lo + j * 16
        pltpu.sync_copy(
            part_hbm.at[0, pl.ds(f * grow + r0, 16)], a0.at[pl.ds(r0, 16)]
        )
        pltpu.sync_copy(
            part_hbm.at[1, pl.ds(f * grow + r0, 16)], a1.at[pl.ds(r0, 16)]
        )
        return carry

      lax.fori_loop(0, ndma, dj, 0)

    ld_rows(n_feat)

    def ci(j, carry):
      r0 = row_lo + j * 16
      for k in range(16):
        cv = a0[r0 + k] + a1[r0 + k]
        icnt[r0 + k] = 1.0 / jnp.maximum(cv, 1.0)
      return carry

    lax.fori_loop(0, ndma, ci, 0)

    def fbody(f, carry):
      ld_rows(f)

      def pr(j, c2):
        r0 = row_lo + j * 16
        for k in range(16):
          pooled[r0 + k] = (a0[r0 + k] + a1[r0 + k]) * icnt[r0 + k]
        return c2

      lax.fori_loop(0, ndma, pr, 0)

      def gb(i, c2):
        for k in range(UNROLL):
          off = (i * UNROLL + k) * LANES
          idxv = idxb[pl.ds(off, LANES)]
          o = plsc.load_gather(
              pooled,
              [lax.shift_right_logical(idxv, 4), lax.bitwise_and(idxv, 15)],
          )
          outb[pl.ds(off, LANES)] = o
        return c2

      lax.fori_loop(0, nv // UNROLL, gb, 0)
      pltpu.sync_copy(outb, out_hbm.at[f, pl.ds(base, chunk)])
      return carry

    lax.fori_loop(0, n_feat, fbody, 0)

  unpool = pl.kernel(
      unpool_body,
      out_type=jax.ShapeDtypeStruct((n_feat, n_ped), jnp.float32),
      mesh=mesh,
      scratch_types=[
          pltpu.VMEM((chunk,), jnp.int32),
          pltpu.VMEM((nrow_loc, LANES), jnp.float32),
          pltpu.VMEM((nrow_loc, LANES), jnp.float32),
          pltpu.VMEM((nrow_loc, LANES), jnp.float32),
          pltpu.VMEM((nrow_loc, LANES), jnp.float32),
          pltpu.VMEM((chunk,), jnp.float32),
      ],
      compiler_params=pltpu.CompilerParams(use_tc_tiling_on_sc=False, needs_layout_passes=False),
      interpret=interpret,
  )

  def run(v, indices):
    n_feat_v = v.shape[1] * v.shape[2]
    v2d = v.reshape(n_feat_v, v.shape[3])
    idx = indices.astype(jnp.int32)
    part = pool(v2d, idx)
    out2d = unpool(part, idx)
    return out2d.reshape(v.shape)

  return run


_run = _build(N_PED, N_GROUPS, N_FEAT)


@jax.jit
def kernel(v, indices):
  return _run(v, indices)
